# baseline (speedup 1.0000x reference)
import functools

import jax
import jax.numpy as jnp
from jax import lax
from jax.experimental import pallas as pl
from jax.experimental.pallas import tpu as pltpu
from jax.experimental.pallas import tpu_sc as plsc

THRESH = 0.2
MIN_KEPT_FRAC = 0.01
ABS_THRESH = 0.8472978603872037

N = 8 * 512 * 512
OUT_ROWS = N // 128

HROWS, HCOLS = 72, 128
H = HROWS * HCOLS
NBUCKETS = 8192
BUCKET_SHIFT = 17
TRASH = NBUCKETS

NTILES = 32
NSPLIT = 4
PART_N = N // NSPLIT
PER_TILE = PART_N // NTILES
CHUNK = 8192

K_KEPT = int(MIN_KEPT_FRAC * N)

PACK = 4096.0


_SOFTPLUS_COEFFS = (
    0.6931471710278221,
    -0.4999994142888598,
    0.12499137715952302,
    5.171521911596488e-05,
    -0.005358852851406005,
    0.0002188750327285275,
    0.000211068980896383,
)


def _elemwise_body(s_ref, t_ref, id_ref):
    s = s_ref[0]
    t = t_ref[0]
    a = jnp.abs(s)
    sp = jnp.float32(_SOFTPLUS_COEFFS[-1])
    for coef in _SOFTPLUS_COEFFS[-2::-1]:
        sp = sp * a + jnp.float32(coef)
    loss = jnp.maximum(s, 0.0) - s * t + sp
    mask = a < ABS_THRESH
    bits = lax.bitcast_convert_type(a, jnp.int32)
    bucket = jnp.minimum(lax.shift_right_logical(bits, BUCKET_SHIFT), NBUCKETS - 1)
    lq = (loss * 65536.0 + 0.5).astype(jnp.int32)
    word = lax.shift_left(jnp.where(mask, bucket, TRASH), 17)
    id_ref[...] = word | jnp.where(mask, lq, 0)


NCHUNK = PER_TILE // CHUNK


def _hist_body(word_hbm, hist_out, w_v0, w_v1, hist_h, sem0, sem1):
    c = lax.axis_index("c")
    s = lax.axis_index("s")
    wid = s * 2 + c
    base = wid * PER_TILE

    zeros16 = jnp.zeros((16,), jnp.float32)

    word_bufs = (w_v0, w_v1)
    sems = (sem0, sem1)

    def issue(ci, b):
        off = base + ci * CHUNK
        pltpu.async_copy(word_hbm.at[pl.ds(off, CHUNK)], word_bufs[b], sems[b])

    def drain(ci, b):
        off = base + ci * CHUNK
        pltpu.make_async_copy(word_hbm.at[pl.ds(off, CHUNK)], word_bufs[b], sems[b]).wait()

    def process(b):
        w_v = word_bufs[b]

        @pl.loop(0, CHUNK // 64)
        def _vec(j):
            for u in range(4):
                o = (j * 4 + u) * 16
                w = w_v[pl.ds(o, 16)]
                idx = lax.shift_right_logical(w, 17)
                lq = w & 0x1FFFF
                v = lq.astype(jnp.float32) * (1.0 / 65536.0) + PACK
                plsc.addupdate_scatter(hist_h, [idx], v)

    issue(0, 0)
    issue(1, 1)

    @pl.loop(0, H // 16)
    def _zero(i):
        hist_h[pl.ds(i * 16, 16)] = zeros16

    @pl.loop(0, NCHUNK - 2, step=2)
    def _outer(ci):
        for b in range(2):
            drain(ci + b, b)
            process(b)
            issue(ci + b + 2, b)

    drain(NCHUNK - 2, 0)
    process(0)
    drain(NCHUNK - 1, 1)
    process(1)

    pltpu.sync_copy(hist_h, hist_out.at[pl.ds(wid * H, H)])


def _masked_roll_add(x, sh, axis, pos):
    return x + jnp.where(pos >= sh, pltpu.roll(x, sh, axis), 0.0)


def _select_body(*args):
    hist_refs, out_ref = args[:-1], args[-1]
    acc_c = jnp.zeros((HROWS, HCOLS), jnp.float32)
    acc_s = jnp.zeros((HROWS, HCOLS), jnp.float32)
    for hist_ref in hist_refs:
        for w in range(NTILES):
            x = hist_ref[pl.ds(w * HROWS, HROWS), :]
            c = jnp.floor(x * (1.0 / PACK))
            acc_c += c
            acc_s += x - PACK * c

    lane = lax.broadcasted_iota(jnp.int32, (HROWS, HCOLS), 1)
    row = lax.broadcasted_iota(jnp.int32, (HROWS, HCOLS), 0)
    valid = (row * HCOLS + lane) < NBUCKETS
    cnt = jnp.where(valid, acc_c, 0.0)
    lsum = jnp.where(valid, acc_s, 0.0)

    x = cnt
    for sh in (1, 2, 4, 8, 16, 32, 64):
        x = _masked_roll_add(x, sh, 1, lane)
    rowt = jnp.broadcast_to(x[:, HCOLS - 1:HCOLS], (HROWS, HCOLS))
    z = rowt
    for sh in (1, 2, 4, 8, 16, 32, 64):
        z = _masked_roll_add(z, sh, 0, row)
    excl = (x + (z - rowt)) - cnt

    total = jnp.sum(cnt)
    k = jnp.minimum(jnp.float32(K_KEPT), total)
    take = jnp.clip(k - excl, 0.0, cnt)
    num = jnp.sum(lsum * take / jnp.maximum(cnt, 1.0))
    out_ref[...] = jnp.reshape(num / jnp.maximum(k, 1.0), (1, 1))


def _make_elemwise(img_off):
    return pl.pallas_call(
        _elemwise_body,
        grid=(32 // NSPLIT,),
        in_specs=[
            pl.BlockSpec((1, 512, 128), lambda i: (i // 4 + img_off, 0, i % 4)),
            pl.BlockSpec((1, 512, 128), lambda i: (i // 4 + img_off, 0, i % 4)),
        ],
        out_specs=pl.BlockSpec((512, 128), lambda i: (i, 0)),
        out_shape=jax.ShapeDtypeStruct((OUT_ROWS // NSPLIT, 128), jnp.int32),
    )


def _make_hist():
    mesh = plsc.VectorSubcoreMesh(core_axis_name="c", subcore_axis_name="s")
    return pl.kernel(
        _hist_body,
        out_type=jax.ShapeDtypeStruct((NTILES * H,), jnp.float32),
        mesh=mesh,
        scratch_types=[
            pltpu.VMEM((CHUNK,), jnp.int32),
            pltpu.VMEM((CHUNK,), jnp.int32),
            pltpu.VMEM((H,), jnp.float32),
            pltpu.SemaphoreType.DMA,
            pltpu.SemaphoreType.DMA,
        ],
        compiler_params=pltpu.CompilerParams(needs_layout_passes=False),
    )


def _make_select():
    return pl.pallas_call(
        _select_body,
        out_shape=jax.ShapeDtypeStruct((1, 1), jnp.float32),
    )


def kernel(score, target):
    hist_call = _make_hist()
    hists = []
    for part in range(NSPLIT):
        words = _make_elemwise(part * (8 // NSPLIT))(score, target)
        hists.append(hist_call(words.reshape(PART_N)))
    out = _make_select()(
        *(h.reshape(NTILES * HROWS, HCOLS) for h in hists)
    )
    return out.reshape(())


# speedup vs baseline: 1.0949x; 1.0949x over previous
import functools

import jax
import jax.numpy as jnp
from jax import lax
from jax.experimental import pallas as pl
from jax.experimental.pallas import tpu as pltpu
from jax.experimental.pallas import tpu_sc as plsc

THRESH = 0.2
MIN_KEPT_FRAC = 0.01
ABS_THRESH = 0.8472978603872037

N = 8 * 512 * 512
OUT_ROWS = N // 128

HROWS, HCOLS = 72, 128
H = HROWS * HCOLS
NBUCKETS = 8192
BUCKET_SHIFT = 17
TRASH = NBUCKETS

NTILES = 32
HALF_N = N // 2
PER_TILE = HALF_N // NTILES
CHUNK = 8192

K_KEPT = int(MIN_KEPT_FRAC * N)

PACK = 4096.0


_SOFTPLUS_COEFFS = (
    0.6931471710278221,
    -0.4999994142888598,
    0.12499137715952302,
    5.171521911596488e-05,
    -0.005358852851406005,
    0.0002188750327285275,
    0.000211068980896383,
)


def _elemwise_body(s_ref, t_ref, id_ref):
    s = s_ref[0]
    t = t_ref[0]
    a = jnp.abs(s)
    sp = jnp.float32(_SOFTPLUS_COEFFS[-1])
    for coef in _SOFTPLUS_COEFFS[-2::-1]:
        sp = sp * a + jnp.float32(coef)
    loss = jnp.maximum(s, 0.0) - s * t + sp
    mask = a < ABS_THRESH
    bits = lax.bitcast_convert_type(a, jnp.int32)
    bucket = jnp.minimum(lax.shift_right_logical(bits, BUCKET_SHIFT), NBUCKETS - 1)
    lq = (loss * 65536.0 + 0.5).astype(jnp.int32)
    word = lax.shift_left(jnp.where(mask, bucket, TRASH), 17)
    id_ref[...] = word | jnp.where(mask, lq, 0)


NCHUNK = PER_TILE // CHUNK


def _hist_body(word_hbm, hist_out, w_v0, w_v1, hist_h, sem0, sem1):
    c = lax.axis_index("c")
    s = lax.axis_index("s")
    wid = s * 2 + c
    base = wid * PER_TILE

    zeros16 = jnp.zeros((16,), jnp.float32)

    word_bufs = (w_v0, w_v1)
    sems = (sem0, sem1)

    def issue(ci, b):
        off = base + ci * CHUNK
        pltpu.async_copy(word_hbm.at[pl.ds(off, CHUNK)], word_bufs[b], sems[b])

    def drain(ci, b):
        off = base + ci * CHUNK
        pltpu.make_async_copy(word_hbm.at[pl.ds(off, CHUNK)], word_bufs[b], sems[b]).wait()

    def process(b):
        w_v = word_bufs[b]

        @pl.loop(0, CHUNK // 64)
        def _vec(j):
            for u in range(4):
                o = (j * 4 + u) * 16
                w = w_v[pl.ds(o, 16)]
                idx = lax.shift_right_logical(w, 17)
                lq = w & 0x1FFFF
                v = lq.astype(jnp.float32) * (1.0 / 65536.0) + PACK
                plsc.addupdate_scatter(hist_h, [idx], v)

    issue(0, 0)
    issue(1, 1)

    @pl.loop(0, H // 16)
    def _zero(i):
        hist_h[pl.ds(i * 16, 16)] = zeros16

    @pl.loop(0, NCHUNK - 2, step=2)
    def _outer(ci):
        for b in range(2):
            drain(ci + b, b)
            process(b)
            issue(ci + b + 2, b)

    drain(NCHUNK - 2, 0)
    process(0)
    drain(NCHUNK - 1, 1)
    process(1)

    pltpu.sync_copy(hist_h, hist_out.at[pl.ds(wid * H, H)])


def _masked_roll_add(x, sh, axis, pos):
    return x + jnp.where(pos >= sh, pltpu.roll(x, sh, axis), 0.0)


def _select_body(hist0_ref, hist1_ref, out_ref):
    acc_c = jnp.zeros((HROWS, HCOLS), jnp.float32)
    acc_s = jnp.zeros((HROWS, HCOLS), jnp.float32)
    for hist_ref in (hist0_ref, hist1_ref):
        for w in range(NTILES):
            x = hist_ref[pl.ds(w * HROWS, HROWS), :]
            c = jnp.floor(x * (1.0 / PACK))
            acc_c += c
            acc_s += x - PACK * c

    lane = lax.broadcasted_iota(jnp.int32, (HROWS, HCOLS), 1)
    row = lax.broadcasted_iota(jnp.int32, (HROWS, HCOLS), 0)
    valid = (row * HCOLS + lane) < NBUCKETS
    cnt = jnp.where(valid, acc_c, 0.0)
    lsum = jnp.where(valid, acc_s, 0.0)

    x = cnt
    for sh in (1, 2, 4, 8, 16, 32, 64):
        x = _masked_roll_add(x, sh, 1, lane)
    rowt = jnp.broadcast_to(x[:, HCOLS - 1:HCOLS], (HROWS, HCOLS))
    z = rowt
    for sh in (1, 2, 4, 8, 16, 32, 64):
        z = _masked_roll_add(z, sh, 0, row)
    excl = (x + (z - rowt)) - cnt

    total = jnp.sum(cnt)
    k = jnp.minimum(jnp.float32(K_KEPT), total)
    take = jnp.clip(k - excl, 0.0, cnt)
    num = jnp.sum(lsum * take / jnp.maximum(cnt, 1.0))
    out_ref[...] = jnp.reshape(num / jnp.maximum(k, 1.0), (1, 1))


def _make_elemwise(img_off):
    return pl.pallas_call(
        _elemwise_body,
        grid=(16,),
        in_specs=[
            pl.BlockSpec((1, 512, 128), lambda i: (i // 4 + img_off, 0, i % 4)),
            pl.BlockSpec((1, 512, 128), lambda i: (i // 4 + img_off, 0, i % 4)),
        ],
        out_specs=pl.BlockSpec((512, 128), lambda i: (i, 0)),
        out_shape=jax.ShapeDtypeStruct((OUT_ROWS // 2, 128), jnp.int32),
    )


def _make_hist():
    mesh = plsc.VectorSubcoreMesh(core_axis_name="c", subcore_axis_name="s")
    return pl.kernel(
        _hist_body,
        out_type=jax.ShapeDtypeStruct((NTILES * H,), jnp.float32),
        mesh=mesh,
        scratch_types=[
            pltpu.VMEM((CHUNK,), jnp.int32),
            pltpu.VMEM((CHUNK,), jnp.int32),
            pltpu.VMEM((H,), jnp.float32),
            pltpu.SemaphoreType.DMA,
            pltpu.SemaphoreType.DMA,
        ],
        compiler_params=pltpu.CompilerParams(needs_layout_passes=False),
    )


def _make_select():
    return pl.pallas_call(
        _select_body,
        out_shape=jax.ShapeDtypeStruct((1, 1), jnp.float32),
    )


def kernel(score, target):
    elem0 = _make_elemwise(0)
    elem1 = _make_elemwise(4)
    hist_call = _make_hist()
    words0 = elem0(score, target)
    hist0 = hist_call(words0.reshape(HALF_N))
    words1 = elem1(score, target)
    hist1 = hist_call(words1.reshape(HALF_N))
    out = _make_select()(
        hist0.reshape(NTILES * HROWS, HCOLS),
        hist1.reshape(NTILES * HROWS, HCOLS),
    )
    return out.reshape(())
